# baseline (speedup 1.0000x reference)
import functools

import jax
import jax.numpy as jnp
from jax.experimental import pallas as pl

N = 10000
E = 320000
F = 128
NH = 128
NC = 10
B = 16

NPAD = 10240


def _mm_kernel(a_ref, w_ref, o_ref):
    o_ref[...] = jnp.dot(a_ref[...], w_ref[...], preferred_element_type=jnp.float32)


def _matmul(a, w):
    m, k = a.shape
    ko = w.shape[1]
    bm = 512
    grid = (m // bm,)
    return pl.pallas_call(
        _mm_kernel,
        grid=grid,
        in_specs=[
            pl.BlockSpec((bm, k), lambda i: (i, 0)),
            pl.BlockSpec((k, ko), lambda i: (0, 0)),
        ],
        out_specs=pl.BlockSpec((bm, ko), lambda i: (i, 0)),
        out_shape=jax.ShapeDtypeStruct((m, ko), jnp.float32),
    )(a, w)


def _sparsemax_bisect(w, row, n, iters=28):
    wmax = jax.ops.segment_max(w, row, num_segments=n)
    lo = wmax - 1.0
    hi = wmax
    for _ in range(iters):
        mid = 0.5 * (lo + hi)
        f = jax.ops.segment_sum(jnp.maximum(w - mid[row], 0.0), row, num_segments=n) - 1.0
        pos = f > 0.0
        lo = jnp.where(pos, mid, lo)
        hi = jnp.where(pos, hi, mid)
    tau = 0.5 * (lo + hi)
    return jnp.maximum(w - tau[row], 0.0)


def kernel(x, edge_index, edge_attr, batch, att1, Wa1, ba1, Wb1, bb1, att2, Wa2, ba2, Wb2, bb2, att3, Wa3, ba3, Wb3, bb3, fcW1, fcb1, fcW2, fcb2, fcW3, fcb3):
    row, col = edge_index[0], edge_index[1]
    xp = jnp.pad(x, ((0, NPAD - N), (0, 0)))

    att_cat = jnp.stack([att1[0, :F], att1[0, F:], att2[0, :F], att2[0, F:],
                         att3[0, :F], att3[0, F:]], axis=1)
    big_w = jnp.concatenate([att_cat, Wa1, Wa2, Wa3], axis=1)
    big = _matmul(xp, big_w)[:N]
    st = big[:, :6]
    xws = (big[:, 6:134], big[:, 134:262], big[:, 262:390])

    feats = []
    for b, (Wb, ba, bb) in enumerate(((Wb1, ba1, bb1), (Wb2, ba2, bb2), (Wb3, ba3, bb3))):
        s = st[:, 2 * b]
        t = st[:, 2 * b + 1]
        w = jax.nn.leaky_relu(s[row] + t[col], 0.2)
        ea = _sparsemax_bisect(w, row, N)
        deg = jnp.ones((N,), jnp.float32).at[col].add(ea)
        dis = jax.lax.rsqrt(deg)
        norm = dis[row] * ea * dis[col]
        self_w = dis * dis
        xw = xws[b]
        h = jnp.zeros_like(xw).at[col].add(xw[row] * norm[:, None]) + self_w[:, None] * xw + ba
        h = jax.nn.relu(h)
        hw = _matmul(jnp.pad(h, ((0, NPAD - N), (0, 0))), Wb)[:N]
        z = jnp.zeros_like(hw).at[col].add(hw[row] * norm[:, None]) + self_w[:, None] * hw + bb
        z = jax.nn.relu(z)
        ones = jnp.ones((N, 1), jnp.float32)
        cnt = jax.ops.segment_sum(ones, batch, num_segments=B)
        mean = jax.ops.segment_sum(z, batch, num_segments=B) / jnp.maximum(cnt, 1.0)
        mx = jax.ops.segment_max(z, batch, num_segments=B)
        mx = jnp.where(cnt > 0, mx, 0.0)
        feats += [mean, mx]

    h = jnp.concatenate(feats, axis=1)
    h = jax.nn.relu(h @ fcW1 + fcb1)
    h = jax.nn.relu(h @ fcW2 + fcb2)
    return jax.nn.log_softmax(h @ fcW3 + fcb3, axis=-1)


# speedup vs baseline: 6.9087x; 6.9087x over previous
import functools

import jax
import jax.numpy as jnp
from jax import lax
from jax.experimental import pallas as pl
from jax.experimental.pallas import tpu as pltpu
from jax.experimental.pallas import tpu_sc as plsc

N = 10000
E = 320000
F = 128
NH = 128
NC = 10
B = 16

NPAD = 10240
NTILES = 32
RPT = 320
CAP = 12288
NCH = CAP // 128
NGRP16 = CAP // 16
MAXD = 80
ELLSZ = (MAXD + 1) * RPT
DUMMY = MAXD * RPT
GROUPS = RPT // 16
KE = 2000
NSCAN = E // KE
BISECT_ITERS = 26
NEG = -1e30


def _mesh():
    return plsc.VectorSubcoreMesh(
        core_axis_name="c", subcore_axis_name="s", num_cores=2, num_subcores=16
    )


def _sc_params():
    return pltpu.CompilerParams(needs_layout_passes=False)


def _wid():
    return lax.axis_index("s") * 2 + lax.axis_index("c")


def _memset(ref, n16, val, dtype):
    v = jnp.full((16,), val, dtype)

    def body(i, _):
        ref[pl.ds(i * 16, 16)] = v
        return 0

    lax.fori_loop(0, n16, body, 0)


def _k1_body(erow, ecol, rows_o, cols_o, pos_o, meta_o,
             rbufA, cbufA, rbufB, cbufB, myrows, mycols, mycols2, mypos,
             metabuf, deg_sm, semA, semB):
    w = _wid()
    lo = w * RPT

    _memset(myrows, CAP // 16, 0, jnp.int32)
    _memset(mycols, CAP // 16, 0, jnp.int32)
    _memset(mypos, CAP // 16, DUMMY, jnp.int32)
    _memset(metabuf, 3, 0, jnp.int32)

    pltpu.async_copy(erow.at[pl.ds(0, KE)], rbufA, semA)
    pltpu.async_copy(ecol.at[pl.ds(0, KE)], cbufA, semA)

    def process(rbuf, cbuf, cursor):
        def grp(i, cur):
            r = rbuf[pl.ds(i * 16, 16)]
            c = cbuf[pl.ds(i * 16, 16)]
            rl = r - lo
            m = (rl >= 0) & (rl < RPT)
            mi = m.astype(jnp.int32)
            cs = plsc.cumsum(mi)
            idx = cur + cs - mi
            plsc.store_scatter(myrows, [idx], rl, mask=m)
            plsc.store_scatter(mycols, [idx], c, mask=m)
            return cur + cs[15]

        return lax.fori_loop(0, KE // 16, grp, cursor)

    def pair(g, cursor):
        offB = (2 * g + 1) * KE
        dB0 = pltpu.async_copy(erow.at[pl.ds(offB, KE)], rbufB, semB)
        dB1 = pltpu.async_copy(ecol.at[pl.ds(offB, KE)], cbufB, semB)
        pltpu.make_async_copy(erow.at[pl.ds(0, KE)], rbufA, semA).wait()
        pltpu.make_async_copy(ecol.at[pl.ds(0, KE)], cbufA, semA).wait()
        cursor = process(rbufA, cbufA, cursor)
        offA = jnp.minimum((2 * g + 2), NSCAN - 1) * KE
        pltpu.async_copy(erow.at[pl.ds(offA, KE)], rbufA, semA)
        pltpu.async_copy(ecol.at[pl.ds(offA, KE)], cbufA, semA)
        dB0.wait()
        dB1.wait()
        cursor = process(rbufB, cbufB, cursor)
        return cursor

    cnt = lax.fori_loop(0, NSCAN // 2, pair, jnp.int32(0))
    pltpu.make_async_copy(erow.at[pl.ds(0, KE)], rbufA, semA).wait()
    pltpu.make_async_copy(ecol.at[pl.ds(0, KE)], cbufA, semA).wait()

    def zdeg(i, _):
        deg_sm[i] = 0
        return 0

    lax.fori_loop(0, RPT, zdeg, 0)
    lane = lax.iota(jnp.int32, 16)
    ngrp = (cnt + 15) // 16

    def ellgrp(i, md):
        r16 = myrows[pl.ds(i * 16, 16)]
        posv = jnp.full((16,), DUMMY, jnp.int32)
        for k in range(16):
            r = r16[k]
            pred = i * 16 + k < cnt
            d = deg_sm[r]
            nd = jnp.where(pred, d + 1, d)
            deg_sm[r] = nd
            posv = jnp.where(lane == k, jnp.where(pred, d * RPT + r, DUMMY), posv)
            md = jnp.maximum(md, nd)
        mypos[pl.ds(i * 16, 16)] = posv
        return md

    md = lax.fori_loop(0, ngrp, ellgrp, jnp.int32(0))

    mv0 = jnp.zeros((16,), jnp.int32)
    mv0 = jnp.where(lane == 0, cnt, mv0)
    mv0 = jnp.where(lane == 1, md, mv0)
    metabuf[pl.ds(0, 16)] = mv0
    gms = []
    for g in range(GROUPS):
        def inner(k, gm, g=g):
            return jnp.maximum(gm, deg_sm[g * 16 + k])

        gms.append(lax.fori_loop(0, 16, inner, jnp.int32(0)))
    for base, lim in ((0, 16), (16, GROUPS - 16)):
        mv = jnp.zeros((16,), jnp.int32)
        for k in range(lim):
            mv = jnp.where(lane == k, gms[base + k], mv)
        metabuf[pl.ds(16 + base, 16)] = mv

    def c2d(i, _):
        v = mycols[pl.ds(i * 16, 16)]
        mycols2[i // 8, pl.ds((i % 8) * 16, 16)] = v
        return 0

    lax.fori_loop(0, NGRP16, c2d, 0)

    pltpu.sync_copy(myrows, rows_o.at[pl.ds(w * CAP, CAP)])
    pltpu.sync_copy(mycols2, cols_o.at[w])
    pltpu.sync_copy(mypos, pos_o.at[pl.ds(w * CAP, CAP)])
    pltpu.sync_copy(metabuf, meta_o.at[pl.ds(w * 48, 48)])


def _run_k1(erow, ecol):
    k1 = pl.kernel(
        _k1_body,
        out_type=(
            jax.ShapeDtypeStruct((NTILES * CAP,), jnp.int32),
            jax.ShapeDtypeStruct((NTILES, NCH, 128), jnp.int32),
            jax.ShapeDtypeStruct((NTILES * CAP,), jnp.int32),
            jax.ShapeDtypeStruct((NTILES * 48,), jnp.int32),
        ),
        mesh=_mesh(),
        compiler_params=_sc_params(),
        scratch_types=[
            pltpu.VMEM((KE,), jnp.int32),
            pltpu.VMEM((KE,), jnp.int32),
            pltpu.VMEM((KE,), jnp.int32),
            pltpu.VMEM((KE,), jnp.int32),
            pltpu.VMEM((CAP,), jnp.int32),
            pltpu.VMEM((CAP,), jnp.int32),
            pltpu.VMEM((NCH, 128), jnp.int32),
            pltpu.VMEM((CAP,), jnp.int32),
            pltpu.VMEM((48,), jnp.int32),
            pltpu.SMEM((RPT,), jnp.int32),
            pltpu.SemaphoreType.DMA,
            pltpu.SemaphoreType.DMA,
        ],
    )
    return k1(erow, ecol)


def _meta_to_smem(metabuf, meta_sm):
    for base in (0, 16, 32):
        v = metabuf[pl.ds(base, 16)]
        for k in range(16):
            meta_sm[base + k] = v[k]


def _k2_body(st, rows_h, cols_h, pos_h, meta_h,
             ea_o, degp_o,
             rows, cols2, pos, wbuf, ell, s_loc, t_full, ea2, tau, metabuf,
             meta_sm, shared_deg):
    w = _wid()
    lo = w * RPT
    sid = lax.axis_index("s")

    pltpu.sync_copy(rows_h.at[pl.ds(w * CAP, CAP)], rows)
    pltpu.sync_copy(cols_h.at[w], cols2)
    pltpu.sync_copy(pos_h.at[pl.ds(w * CAP, CAP)], pos)
    pltpu.sync_copy(meta_h.at[pl.ds(w * 48, 48)], metabuf)
    _meta_to_smem(metabuf, meta_sm)
    cnt = meta_sm[0]
    ngrp = (cnt + 15) // 16
    ncc = (cnt + 127) // 128

    def block(b, _):
        @pl.when(sid == 0)
        def _():
            _memset(t_full, NPAD // 16, 0.0, jnp.float32)
            pltpu.sync_copy(t_full, shared_deg)

        plsc.subcore_barrier()

        pltpu.sync_copy(st.at[pl.ds(2 * b * NPAD + lo, RPT)], s_loc)
        pltpu.sync_copy(st.at[pl.ds((2 * b + 1) * NPAD, NPAD)], t_full)
        _memset(ell, ELLSZ // 16, NEG, jnp.float32)

        lane = lax.iota(jnp.int32, 16)

        def score(i, _):
            r = rows[pl.ds(i * 16, 16)]
            c = cols2[i // 8, pl.ds((i % 8) * 16, 16)]
            sv = plsc.load_gather(s_loc, [r])
            tv = plsc.load_gather(t_full, [c])
            wv = sv + tv
            wv = jnp.where(wv > 0, wv, 0.2 * wv)
            wv = jnp.where(lane + i * 16 < cnt, wv, NEG)
            wbuf[pl.ds(i * 16, 16)] = wv
            pv = pos[pl.ds(i * 16, 16)]
            plsc.store_scatter(ell, [pv], wv)
            return 0

        lax.fori_loop(0, ngrp, score, 0)

        for g in range(GROUPS):
            gm = meta_sm[16 + g]
            base = g * 16

            def wmax_j(j, acc, base=base):
                return jnp.maximum(acc, ell[pl.ds(j * RPT + base, 16)])

            wmax = lax.fori_loop(0, gm, wmax_j, jnp.full((16,), NEG, jnp.float32))

            def bis(_, lh, gm=gm, base=base):
                lov, hiv = lh
                mid = 0.5 * (lov + hiv)

                def fsum(j, acc, base=base, mid=mid):
                    return acc + jnp.maximum(ell[pl.ds(j * RPT + base, 16)] - mid, 0.0)

                fs = lax.fori_loop(0, gm, fsum, jnp.full((16,), -1.0, jnp.float32))
                pos_m = fs > 0
                return (jnp.where(pos_m, mid, lov), jnp.where(pos_m, hiv, mid))

            lov, hiv = lax.fori_loop(0, BISECT_ITERS, bis, (wmax - 1.0, wmax))
            tau[pl.ds(base, 16)] = 0.5 * (lov + hiv)

        def eal(i, _):
            wv = wbuf[pl.ds(i * 16, 16)]
            r = rows[pl.ds(i * 16, 16)]
            tv = plsc.load_gather(tau, [r])
            ea2[i // 8, pl.ds((i % 8) * 16, 16)] = jnp.maximum(wv - tv, 0.0)
            return 0

        lax.fori_loop(0, ngrp, eal, 0)
        def ztail(i, _):
            ea2[i // 8, pl.ds((i % 8) * 16, 16)] = jnp.zeros((16,), jnp.float32)
            return 0

        lax.fori_loop(ngrp, NGRP16, ztail, 0)

        pltpu.sync_copy(ea2, ea_o.at[b, w])

        def degscat(j, _):
            pltpu.sync_copy(ea2.at[j], shared_deg.at[cols2.at[j]], add=True)
            return 0

        lax.fori_loop(0, ncc, degscat, 0)
        plsc.subcore_barrier()

        @pl.when(sid == 0)
        def _():
            pltpu.sync_copy(
                shared_deg,
                degp_o.at[pl.ds((b * 2 + lax.axis_index("c")) * NPAD, NPAD)])

        plsc.subcore_barrier()
        return 0

    lax.fori_loop(0, 3, block, 0)


def _run_k2(st_t, rows, cols, pos, meta):
    k2 = pl.kernel(
        _k2_body,
        out_type=(
            jax.ShapeDtypeStruct((3, NTILES, NCH, 128), jnp.float32),
            jax.ShapeDtypeStruct((3 * 2 * NPAD,), jnp.float32),
        ),
        mesh=_mesh(),
        compiler_params=_sc_params(),
        scratch_types=[
            pltpu.VMEM((CAP,), jnp.int32),
            pltpu.VMEM((NCH, 128), jnp.int32),
            pltpu.VMEM((CAP,), jnp.int32),
            pltpu.VMEM((CAP,), jnp.float32),
            pltpu.VMEM((ELLSZ,), jnp.float32),
            pltpu.VMEM((RPT,), jnp.float32),
            pltpu.VMEM((NPAD,), jnp.float32),
            pltpu.VMEM((NCH, 128), jnp.float32),
            pltpu.VMEM((RPT,), jnp.float32),
            pltpu.VMEM((48,), jnp.int32),
            pltpu.SMEM((48,), jnp.int32),
            pltpu.VMEM_SHARED((NPAD,), jnp.float32),
        ],
    )
    return k2(st_t, rows, cols, pos, meta)


def _mm_kernel(a_ref, w_ref, o_ref):
    o_ref[...] = jnp.dot(a_ref[...], w_ref[...], preferred_element_type=jnp.float32)


def _matmul(a, w):
    m, k = a.shape
    ko = w.shape[1]
    bm = 512
    return pl.pallas_call(
        _mm_kernel,
        grid=(m // bm,),
        in_specs=[
            pl.BlockSpec((bm, k), lambda i: (i, 0)),
            pl.BlockSpec((k, ko), lambda i: (0, 0)),
        ],
        out_specs=pl.BlockSpec((bm, ko), lambda i: (i, 0)),
        out_shape=jax.ShapeDtypeStruct((m, ko), jnp.float32),
    )(a, w)


def kernel(x, edge_index, edge_attr, batch, att1, Wa1, ba1, Wb1, bb1, att2, Wa2, ba2, Wb2, bb2, att3, Wa3, ba3, Wb3, bb3, fcW1, fcb1, fcW2, fcb2, fcW3, fcb3):
    xp = jnp.pad(x, ((0, NPAD - N), (0, 0)))

    att6 = jnp.stack([att1[0, :F], att1[0, F:], att2[0, :F], att2[0, F:],
                      att3[0, :F], att3[0, F:]], axis=1)
    big_w = jnp.concatenate([att6, Wa1, Wa2, Wa3], axis=1)
    big = _matmul(xp, big_w)
    st_t = jnp.transpose(big[:, :6]).reshape(-1)
    xws = (big[:, 6:134], big[:, 134:262], big[:, 262:390])

    rows, cols, pos, meta = _run_k1(edge_index[0], edge_index[1])
    ea3, degp3 = _run_k2(st_t, rows, cols, pos, meta)
    degp3 = degp3.reshape(3, 2, NPAD)

    tid = jnp.arange(NTILES, dtype=jnp.int32)
    row_g = (rows.reshape(NTILES, CAP) + tid[:, None] * RPT).reshape(-1)
    col_g = cols.reshape(-1)
    feats = []
    for b, (Wb, ba, bb) in enumerate(((Wb1, ba1, bb1), (Wb2, ba2, bb2), (Wb3, ba3, bb3))):
        ea = ea3[b].reshape(-1)
        deg = degp3[b, 0] + degp3[b, 1] + 1.0
        dis = lax.rsqrt(deg)
        norm = dis[row_g] * ea * dis[col_g]
        self_w = (dis * dis)[:N]
        xw = xws[b][:N]
        h = jnp.zeros_like(xw).at[col_g].add(
            xw[jnp.minimum(row_g, N - 1)] * norm[:, None],
            mode="drop") + self_w[:, None] * xw + ba
        h = jax.nn.relu(h)
        hw = _matmul(jnp.pad(h, ((0, NPAD - N), (0, 0))), Wb)[:N]
        z = jnp.zeros_like(hw).at[col_g].add(
            hw[jnp.minimum(row_g, N - 1)] * norm[:, None],
            mode="drop") + self_w[:, None] * hw + bb
        z = jax.nn.relu(z)
        ones = jnp.ones((N, 1), jnp.float32)
        cnt = jax.ops.segment_sum(ones, batch, num_segments=B)
        mean = jax.ops.segment_sum(z, batch, num_segments=B) / jnp.maximum(cnt, 1.0)
        mx = jax.ops.segment_max(z, batch, num_segments=B)
        mx = jnp.where(cnt > 0, mx, 0.0)
        feats += [mean, mx]

    h = jnp.concatenate(feats, axis=1)
    h = jax.nn.relu(h @ fcW1 + fcb1)
    h = jax.nn.relu(h @ fcW2 + fcb2)
    return jax.nn.log_softmax(h @ fcW3 + fcb3, axis=-1)
